# baseline (speedup 1.0000x reference)
import functools

import jax
import jax.numpy as jnp
from jax import lax
from jax.experimental import pallas as pl
from jax.experimental.pallas import tpu as pltpu
from jax.experimental.pallas import tpu_sc as plsc

B, C, D = 1024, 512, 64
NROW = B * C * D // 8192
RB = 256
NC, NS, L = 2, 16, 16
NW = NC * NS
BPW = B // NW


def _cls_body(a_ref, b_ref, cls_ref):
    a = a_ref[...]
    out = jnp.dot(a * a, b_ref[...], preferred_element_type=jnp.float32,
                  precision=lax.Precision.HIGHEST)
    cls_ref[...] = jnp.sqrt(out)


def _classes(xa, bmat):
    return pl.pallas_call(
        _cls_body,
        grid=(NROW // RB,),
        in_specs=[
            pl.BlockSpec((RB, 8192), lambda i: (i, 0)),
            pl.BlockSpec((8192, 128), lambda i: (0, 0)),
        ],
        out_specs=pl.BlockSpec((RB, 128), lambda i: (i, 0)),
        out_shape=jax.ShapeDtypeStruct((NROW, 128), jnp.float32),
    )(xa, bmat)


_mesh = plsc.VectorSubcoreMesh(core_axis_name="c", subcore_axis_name="s")


@functools.partial(
    pl.kernel,
    out_type=[
        jax.ShapeDtypeStruct((B,), jnp.int32),
        jax.ShapeDtypeStruct((B, D), jnp.float32),
    ],
    mesh=_mesh,
    compiler_params=pltpu.CompilerParams(
        needs_layout_passes=False, use_tc_tiling_on_sc=False),
    scratch_types=[
        pltpu.VMEM((BPW * C,), jnp.float32),
        pltpu.VMEM((BPW,), jnp.int32),
        pltpu.VMEM((BPW, D), jnp.float32),
        pltpu.SemaphoreType.DMA,
    ],
)
def _sc_pick(cls_hbm, x_hbm, idx_hbm, masked_hbm, cls_v, idx_v, rows_v, sem):
    wid = lax.axis_index("s") * NC + lax.axis_index("c")
    base = wid * BPW
    pltpu.sync_copy(cls_hbm.at[pl.ds(base * C, BPW * C)], cls_v)
    lane = lax.broadcasted_iota(jnp.int32, (L,), 0)
    for g in range(BPW // L):
        rids = g * L + lane

        def body(c, carry):
            vmax, varg = carry
            v = plsc.load_gather(cls_v, [rids * C + c])
            take = v > vmax
            return jnp.where(take, v, vmax), jnp.where(take, c, varg)

        _, varg = lax.fori_loop(
            0, C, body,
            (jnp.full((L,), -1.0, jnp.float32), jnp.zeros((L,), jnp.int32)),
        )
        idx_v[pl.ds(g * L, L)] = varg
        grow = (base + rids) * C + varg
        pltpu.async_copy(x_hbm.at[grow], rows_v.at[pl.ds(g * L, L)], sem).wait()
    pltpu.sync_copy(idx_v, idx_hbm.at[pl.ds(base, BPW)])
    pltpu.sync_copy(rows_v, masked_hbm.at[pl.ds(base, BPW)])


def kernel(x):
    xa = x.reshape(NROW, 8192)
    bmat = (jnp.arange(8192, dtype=jnp.int32)[:, None] // D
            == jnp.arange(128, dtype=jnp.int32)[None, :]).astype(jnp.float32)
    cls = _classes(xa, bmat).reshape(B, C)
    idx, masked = _sc_pick(cls.reshape(B * C), x.reshape(B * C, D))
    return masked[:, None, :], idx, cls
